# Initial kernel scaffold; baseline (speedup 1.0000x reference)
import functools

import jax
import jax.numpy as jnp
from jax.experimental import pallas as pl
from jax.experimental.pallas import tpu as pltpu
from jax.experimental.pallas import tpu_sc as plsc

HIDDEN = 32
WINDOW = 128


def _gather_rows(table, flat_ids):
    num = flat_ids.shape[1]
    mesh = plsc.VectorSubcoreMesh(core_axis_name="c", subcore_axis_name="s")

    @functools.partial(
        pl.kernel,
        out_type=jax.ShapeDtypeStruct((num, HIDDEN), table.dtype),
        mesh=mesh,
    )
    def gather_kernel(table_hbm, idx_hbm, out_hbm):
        def body(idx_vmem, out_vmem):
            pltpu.sync_copy(table_hbm.at[idx_vmem.at[0]], out_vmem)

        pltpu.emit_pipeline(
            body,
            grid=(num // WINDOW,),
            in_specs=[pl.BlockSpec((1, WINDOW), lambda i: (0, i))],
            out_specs=[pl.BlockSpec((WINDOW, HIDDEN), lambda i: (i, 0))],
            core_axis_name=("c", "s"),
            dimension_semantics=(pltpu.PARALLEL,),
        )(idx_hbm, out_hbm)

    return gather_kernel(table, flat_ids)


def kernel(input_ids, table):
    batch = input_ids.size
    flat_ids = input_ids.reshape(1, batch).astype(jnp.int32)
    out = _gather_rows(table, flat_ids)
    return out.reshape(*input_ids.shape, HIDDEN)


# speedup vs baseline: 1.0211x; 1.0211x over previous
import functools

import jax
import jax.numpy as jnp
from jax import lax
from jax.experimental import pallas as pl
from jax.experimental.pallas import tpu as pltpu
from jax.experimental.pallas import tpu_sc as plsc

HIDDEN = 32
REP = 4
WIDE = HIDDEN * REP
W = 128
NC, NS = 2, 16
NW = NC * NS


def _gather_call(table_wide, idx3):
    nw, nb, w = idx3.shape
    batch = nw * nb * w
    rows_per_worker = nb * w
    mesh = plsc.VectorSubcoreMesh(core_axis_name="c", subcore_axis_name="s")

    @functools.partial(
        pl.kernel,
        out_type=jax.ShapeDtypeStruct((batch, WIDE), table_wide.dtype),
        mesh=mesh,
        scratch_types=[
            pltpu.VMEM((nb, w), jnp.int32),
            pltpu.VMEM((W, WIDE), jnp.float32),
            pltpu.VMEM((W, WIDE), jnp.float32),
            pltpu.SemaphoreType.DMA,
            pltpu.SemaphoreType.DMA,
            pltpu.SemaphoreType.DMA,
            pltpu.SemaphoreType.DMA,
            pltpu.SemaphoreType.DMA,
        ],
    )
    def k(tw_hbm, idx_hbm, out_hbm, idx_v, rb0, rb1, si, sg0, sg1, sw0, sw1):
        wid = lax.axis_index("s") * NC + lax.axis_index("c")
        base = wid * rows_per_worker
        pltpu.async_copy(idx_hbm.at[wid], idx_v, si).wait()

        bufs = (rb0, rb1)
        gsems = (sg0, sg1)
        wsems = (sw0, sw1)

        def gather_start(j, t):
            pltpu.make_async_copy(
                tw_hbm.at[idx_v.at[j]], bufs[t], gsems[t]
            ).start()

        def gather_wait(t):
            pltpu.make_async_copy(
                tw_hbm.at[idx_v.at[0]], bufs[t], gsems[t]
            ).wait()

        def write_start(j, t):
            pltpu.make_async_copy(
                bufs[t],
                out_hbm.at[pl.ds(base + j * W, W)],
                wsems[t],
            ).start()

        def write_wait(t):
            pltpu.make_async_copy(
                bufs[t],
                out_hbm.at[pl.ds(base, W)],
                wsems[t],
            ).wait()

        gather_start(0, 0)
        gather_start(1, 1)

        @pl.loop(0, nb, step=2)
        def _(j):
            for t in range(2):
                gather_wait(t)
                write_start(j + t, t)

            @pl.when(j + 2 < nb)
            def _():
                for t in range(2):
                    write_wait(t)
                    gather_start(j + 2 + t, t)

        for t in range(2):
            write_wait(t)

    return k(table_wide, idx3)


def kernel(input_ids, table):
    batch = input_ids.size
    table_wide = jnp.tile(table, (1, REP))
    idx3 = input_ids.reshape(NW, batch // (NW * W), W).astype(jnp.int32)
    out = _gather_call(table_wide, idx3)
    return out[:, :HIDDEN].reshape(*input_ids.shape, HIDDEN)


# speedup vs baseline: 1.2185x; 1.1933x over previous
import functools

import jax
import jax.numpy as jnp
from jax import lax
from jax.experimental import pallas as pl
from jax.experimental.pallas import tpu as pltpu
from jax.experimental.pallas import tpu_sc as plsc

HIDDEN = 32
CHUNK = 256
NC, NS = 2, 16
NW = NC * NS


def _gather_call(table, idx3):
    nw, nb, w = idx3.shape
    batch = nw * nb * w
    rows_per_worker = nb * w
    mesh = plsc.VectorSubcoreMesh(core_axis_name="c", subcore_axis_name="s")

    @functools.partial(
        pl.kernel,
        out_type=jax.ShapeDtypeStruct((batch, HIDDEN), table.dtype),
        mesh=mesh,
        scratch_types=[
            pltpu.VMEM((2, w), jnp.int32),
            pltpu.VMEM((CHUNK, HIDDEN), jnp.float32),
            pltpu.VMEM((CHUNK, HIDDEN), jnp.float32),
            pltpu.SemaphoreType.DMA,
            pltpu.SemaphoreType.DMA,
            pltpu.SemaphoreType.DMA,
            pltpu.SemaphoreType.DMA,
            pltpu.SemaphoreType.DMA,
        ],
    )
    def k(table_hbm, idx_hbm, out_hbm, idx_vm, rb0, rb1, si0, sg0, sg1, sw0, sw1):
        wid = lax.axis_index("s") * NC + lax.axis_index("c")
        base = wid * rows_per_worker

        bufs = (rb0, rb1)
        gsems = (sg0, sg1)
        wsems = (sw0, sw1)

        def idx_load(j, t):
            pltpu.make_async_copy(
                idx_hbm.at[wid, j], idx_vm.at[t], si0
            ).start()

        def idx_wait(t):
            pltpu.make_async_copy(
                idx_hbm.at[wid, 0], idx_vm.at[t], si0
            ).wait()

        def gather_chunk(t):
            @pl.loop(0, CHUNK, step=16)
            def _(r):
                v = idx_vm[t, pl.ds(r, 16)]
                for l in range(16):
                    pltpu.make_async_copy(
                        table_hbm.at[pl.ds(v[l], 1)],
                        bufs[t].at[pl.ds(r + l, 1)],
                        gsems[t],
                    ).start()

        def gather_drain(t):
            pltpu.make_async_copy(
                table_hbm.at[pl.ds(0, CHUNK)], bufs[t], gsems[t]
            ).wait()

        def write_start(j, t):
            pltpu.make_async_copy(
                bufs[t],
                out_hbm.at[pl.ds(base + j * CHUNK, CHUNK)],
                wsems[t],
            ).start()

        def write_wait(t):
            pltpu.make_async_copy(
                bufs[t],
                out_hbm.at[pl.ds(base, CHUNK)],
                wsems[t],
            ).wait()

        idx_load(0, 0)
        idx_load(1, 1)

        @pl.loop(0, nb, step=2)
        def _(j):
            for t in range(2):
                idx_wait(t)

                @pl.when(j > 0)
                def _():
                    write_wait(t)

                gather_chunk(t)

            for t in range(2):
                gather_drain(t)
                write_start(j + t, t)

            @pl.when(j + 2 < nb)
            def _():
                for t in range(2):
                    idx_load(j + 2 + t, t)

        for t in range(2):
            write_wait(t)

    return k(table, idx3)


def kernel(input_ids, table):
    batch = input_ids.size
    idx3 = input_ids.reshape(NW, batch // (NW * CHUNK), CHUNK).astype(jnp.int32)
    out = _gather_call(table, idx3)
    return out.reshape(*input_ids.shape, HIDDEN)


# speedup vs baseline: 2.0710x; 1.6997x over previous
import functools

import jax
import jax.numpy as jnp
from jax import lax
from jax.experimental import pallas as pl
from jax.experimental.pallas import tpu as pltpu
from jax.experimental.pallas import tpu_sc as plsc

HIDDEN = 32
SEQ = 50
BPC = 8
CHUNK = BPC * SEQ
NC, NS = 2, 16
NW = NC * NS


def _gather_call(table, idx3, n_batch):
    nw, nb, w = idx3.shape
    batches_per_worker = nb * BPC
    mesh = plsc.VectorSubcoreMesh(core_axis_name="c", subcore_axis_name="s")

    @functools.partial(
        pl.kernel,
        out_type=jax.ShapeDtypeStruct((n_batch, SEQ, HIDDEN), table.dtype),
        mesh=mesh,
        scratch_types=[
            pltpu.VMEM((2, w), jnp.int32),
            pltpu.VMEM((CHUNK, HIDDEN), jnp.float32),
            pltpu.VMEM((CHUNK, HIDDEN), jnp.float32),
            pltpu.SemaphoreType.DMA,
            pltpu.SemaphoreType.DMA,
            pltpu.SemaphoreType.DMA,
            pltpu.SemaphoreType.DMA,
            pltpu.SemaphoreType.DMA,
        ],
    )
    def k(table_hbm, idx_hbm, out_hbm, idx_vm, rb0, rb1, si0, sg0, sg1, sw0, sw1):
        wid = lax.axis_index("s") * NC + lax.axis_index("c")
        base_batch = wid * batches_per_worker

        bufs = (rb0, rb1)
        gsems = (sg0, sg1)
        wsems = (sw0, sw1)

        def idx_load(j, t):
            pltpu.make_async_copy(
                idx_hbm.at[wid, j], idx_vm.at[t], si0
            ).start()

        def idx_wait(t):
            pltpu.make_async_copy(
                idx_hbm.at[wid, 0], idx_vm.at[t], si0
            ).wait()

        def gather_chunk(t):
            @pl.loop(0, CHUNK, step=16)
            def _(r):
                v = idx_vm[t, pl.ds(r, 16)]
                for l in range(16):
                    pltpu.make_async_copy(
                        table_hbm.at[pl.ds(v[l], 1)],
                        bufs[t].at[pl.ds(r + l, 1)],
                        gsems[t],
                    ).start()

        def gather_drain(t):
            pltpu.make_async_copy(
                table_hbm.at[pl.ds(0, CHUNK)], bufs[t], gsems[t]
            ).wait()

        def write_start(j, t):
            pltpu.make_async_copy(
                bufs[t].reshape(BPC, SEQ, HIDDEN),
                out_hbm.at[pl.ds(base_batch + j * BPC, BPC)],
                wsems[t],
            ).start()

        def write_wait(t):
            pltpu.make_async_copy(
                bufs[t].reshape(BPC, SEQ, HIDDEN),
                out_hbm.at[pl.ds(base_batch, BPC)],
                wsems[t],
            ).wait()

        idx_load(0, 0)
        idx_load(1, 1)

        @pl.loop(0, nb, step=2)
        def _(j):
            for t in range(2):
                idx_wait(t)

                @pl.when(j > 0)
                def _():
                    write_wait(t)

                gather_chunk(t)

            for t in range(2):
                gather_drain(t)
                write_start(j + t, t)

            @pl.when(j + 2 < nb)
            def _():
                for t in range(2):
                    idx_load(j + 2 + t, t)

        for t in range(2):
            write_wait(t)

    return k(table, idx3)


def kernel(input_ids, table):
    n_batch, seq = input_ids.shape
    idx3 = input_ids.reshape(NW, n_batch * seq // (NW * CHUNK), CHUNK).astype(
        jnp.int32
    )
    return _gather_call(table, idx3, n_batch)
